# Initial kernel scaffold; baseline (speedup 1.0000x reference)
import jax
import jax.numpy as jnp
from jax.experimental import pallas as pl
from jax.experimental.pallas import tpu as pltpu

_E = 64
_K = 8
_ALPHA = 0.01


def _router_block(x_ref, wt_ref, tw_ref, ti_ref, aux_ref, ssum_ref, cnt_ref):
    i = pl.program_id(0)
    n = pl.num_programs(0)

    x = x_ref[...]
    wt = wt_ref[...]
    logits = jnp.dot(x, wt, preferred_element_type=jnp.float32)

    m = jnp.max(logits, axis=1, keepdims=True)
    ex = jnp.exp(logits - m)
    z = jnp.sum(ex, axis=1, keepdims=True)
    scores = ex / z

    iota = jax.lax.broadcasted_iota(jnp.int32, scores.shape, 1)
    work = scores
    ws = []
    idxs = []
    hits = jnp.zeros_like(scores)
    for _ in range(_K):
        mk = jnp.max(work, axis=1, keepdims=True)
        sel = work == mk
        ik = jnp.min(jnp.where(sel, iota, _E), axis=1, keepdims=True)
        picked = iota == ik
        hits = hits + picked.astype(jnp.float32)
        ws.append(mk)
        idxs.append(ik)
        work = jnp.where(picked, -jnp.inf, work)

    tw = jnp.concatenate(ws, axis=1)
    ti = jnp.concatenate(idxs, axis=1)
    tw = tw / (jnp.sum(tw, axis=1, keepdims=True) + 1e-9)

    tw_ref[...] = tw
    ti_ref[...] = ti

    block_ssum = jnp.sum(scores, axis=0, keepdims=True)
    block_cnt = jnp.sum(hits, axis=0, keepdims=True)

    @pl.when(i == 0)
    def _init():
        ssum_ref[...] = block_ssum
        cnt_ref[...] = block_cnt

    @pl.when(i > 0)
    def _acc():
        ssum_ref[...] += block_ssum
        cnt_ref[...] += block_cnt

    @pl.when(i == n - 1)
    def _finish():
        t_total = n * x.shape[0]
        scale = _ALPHA * _E / (float(t_total) * float(t_total) * _K)
        aux_ref[0, 0] = jnp.sum(ssum_ref[...] * cnt_ref[...]) * scale


def kernel(x, W):
    bsz, seq, d = x.shape
    t = bsz * seq
    xf = x.reshape(t, d)
    wt = W.T

    blk = 512
    grid = (t // blk,)

    tw, ti, aux = pl.pallas_call(
        _router_block,
        grid=grid,
        in_specs=[
            pl.BlockSpec((blk, d), lambda i: (i, 0)),
            pl.BlockSpec((d, _E), lambda i: (0, 0)),
        ],
        out_specs=[
            pl.BlockSpec((blk, _K), lambda i: (i, 0)),
            pl.BlockSpec((blk, _K), lambda i: (i, 0)),
            pl.BlockSpec((1, 1), lambda i: (0, 0)),
        ],
        out_shape=[
            jax.ShapeDtypeStruct((t, _K), jnp.float32),
            jax.ShapeDtypeStruct((t, _K), jnp.int32),
            jax.ShapeDtypeStruct((1, 1), jnp.float32),
        ],
        scratch_shapes=[
            pltpu.VMEM((1, _E), jnp.float32),
            pltpu.VMEM((1, _E), jnp.float32),
        ],
    )(xf, wt)

    return tw, ti, aux[0, 0]


# speedup vs baseline: 1.2650x; 1.2650x over previous
import jax
import jax.numpy as jnp
from jax.experimental import pallas as pl
from jax.experimental.pallas import tpu as pltpu

_E = 64
_K = 8
_ALPHA = 0.01


def _router_block(x_ref, wt_ref, tw_ref, ti_ref, aux_ref, ssum_ref, cnt_ref):
    i = pl.program_id(0)
    n = pl.num_programs(0)

    x = x_ref[...]
    wt = wt_ref[...]
    logits = jnp.dot(x, wt, preferred_element_type=jnp.float32)

    m = jnp.max(logits, axis=1, keepdims=True)
    ex = jnp.exp(logits - m)
    z = jnp.sum(ex, axis=1, keepdims=True)
    scores = ex / z

    iota = jax.lax.broadcasted_iota(jnp.int32, scores.shape, 1)
    work = scores
    ws = []
    idxs = []
    hits = jnp.zeros_like(scores)
    for _ in range(_K):
        mk = jnp.max(work, axis=1, keepdims=True)
        sel = work == mk
        ik = jnp.min(jnp.where(sel, iota, _E), axis=1, keepdims=True)
        picked = iota == ik
        hits = hits + picked.astype(jnp.float32)
        ws.append(mk)
        idxs.append(ik)
        work = jnp.where(picked, -jnp.inf, work)

    tw = jnp.concatenate(ws, axis=1)
    ti = jnp.concatenate(idxs, axis=1)
    tw = tw / (jnp.sum(tw, axis=1, keepdims=True) + 1e-9)

    tw_ref[...] = tw
    ti_ref[...] = ti

    block_ssum = jnp.sum(scores, axis=0, keepdims=True)
    block_cnt = jnp.sum(hits, axis=0, keepdims=True)

    @pl.when(i == 0)
    def _init():
        ssum_ref[...] = block_ssum
        cnt_ref[...] = block_cnt

    @pl.when(i > 0)
    def _acc():
        ssum_ref[...] += block_ssum
        cnt_ref[...] += block_cnt

    @pl.when(i == n - 1)
    def _finish():
        t_total = n * x.shape[0]
        scale = _ALPHA * _E / (float(t_total) * float(t_total) * _K)
        aux_ref[...] = jnp.sum(
            ssum_ref[...] * cnt_ref[...], axis=1, keepdims=True
        ) * scale


def kernel(x, W):
    bsz, seq, d = x.shape
    t = bsz * seq
    xf = x.reshape(t, d)
    wt = W.T

    blk = 512
    grid = (t // blk,)

    tw, ti, aux = pl.pallas_call(
        _router_block,
        grid=grid,
        in_specs=[
            pl.BlockSpec((blk, d), lambda i: (i, 0)),
            pl.BlockSpec((d, _E), lambda i: (0, 0)),
        ],
        out_specs=[
            pl.BlockSpec((blk, _K), lambda i: (i, 0)),
            pl.BlockSpec((blk, _K), lambda i: (i, 0)),
            pl.BlockSpec((1, 1), lambda i: (0, 0)),
        ],
        out_shape=[
            jax.ShapeDtypeStruct((t, _K), jnp.float32),
            jax.ShapeDtypeStruct((t, _K), jnp.int32),
            jax.ShapeDtypeStruct((1, 1), jnp.float32),
        ],
        scratch_shapes=[
            pltpu.VMEM((1, _E), jnp.float32),
            pltpu.VMEM((1, _E), jnp.float32),
        ],
    )(xf, wt)

    return tw, ti, aux[0, 0]


# speedup vs baseline: 1.3344x; 1.0549x over previous
import jax
import jax.numpy as jnp
from jax.experimental import pallas as pl
from jax.experimental.pallas import tpu as pltpu

_E = 64
_K = 8
_ALPHA = 0.01


def _router_block(x_ref, wt_ref, tw_ref, ti_ref, aux_ref, ssum_ref, cnt_ref):
    i = pl.program_id(0)
    n = pl.num_programs(0)

    x = x_ref[...]
    wt = wt_ref[...]
    logits = jnp.dot(x, wt, preferred_element_type=jnp.float32)

    m = jnp.max(logits, axis=1, keepdims=True)
    ex = jnp.exp(logits - m)
    z = jnp.sum(ex, axis=1, keepdims=True)
    scores = ex / z

    iota_f = jax.lax.broadcasted_iota(jnp.int32, scores.shape, 1).astype(
        jnp.float32)
    sbits = jax.lax.bitcast_convert_type(scores, jnp.int32)
    work = sbits
    mks = []
    idxs = []
    for _ in range(_K):
        mk = jnp.max(work, axis=1, keepdims=True)
        eq = work == mk
        idxs.append(jnp.sum(jnp.where(eq, iota_f, 0.0), axis=1, keepdims=True))
        work = jnp.where(eq, jnp.int32(-(2**31)), work)
        mks.append(mk)

    mkcat = jnp.concatenate(mks, axis=1)
    ti = jnp.concatenate(idxs, axis=1).astype(jnp.int32)
    tw = jax.lax.bitcast_convert_type(mkcat, jnp.float32)
    tw = tw / (jnp.sum(tw, axis=1, keepdims=True) + 1e-9)
    hits = (sbits >= mks[-1]).astype(jnp.float32)

    tw_ref[...] = tw
    ti_ref[...] = ti

    block_ssum = jnp.sum(scores, axis=0, keepdims=True)
    block_cnt = jnp.sum(hits, axis=0, keepdims=True)

    @pl.when(i == 0)
    def _init():
        ssum_ref[...] = block_ssum
        cnt_ref[...] = block_cnt

    @pl.when(i > 0)
    def _acc():
        ssum_ref[...] += block_ssum
        cnt_ref[...] += block_cnt

    @pl.when(i == n - 1)
    def _finish():
        t_total = n * x.shape[0]
        scale = _ALPHA * _E / (float(t_total) * float(t_total) * _K)
        aux_ref[...] = jnp.sum(
            ssum_ref[...] * cnt_ref[...], axis=1, keepdims=True
        ) * scale


def kernel(x, W):
    bsz, seq, d = x.shape
    t = bsz * seq
    xf = x.reshape(t, d)
    wt = W.T

    blk = 512
    grid = (t // blk,)

    tw, ti, aux = pl.pallas_call(
        _router_block,
        grid=grid,
        in_specs=[
            pl.BlockSpec((blk, d), lambda i: (i, 0)),
            pl.BlockSpec((d, _E), lambda i: (0, 0)),
        ],
        out_specs=[
            pl.BlockSpec((blk, _K), lambda i: (i, 0)),
            pl.BlockSpec((blk, _K), lambda i: (i, 0)),
            pl.BlockSpec((1, 1), lambda i: (0, 0)),
        ],
        out_shape=[
            jax.ShapeDtypeStruct((t, _K), jnp.float32),
            jax.ShapeDtypeStruct((t, _K), jnp.int32),
            jax.ShapeDtypeStruct((1, 1), jnp.float32),
        ],
        scratch_shapes=[
            pltpu.VMEM((1, _E), jnp.float32),
            pltpu.VMEM((1, _E), jnp.float32),
        ],
    )(xf, wt)

    return tw, ti, aux[0, 0]


# speedup vs baseline: 1.8751x; 1.4052x over previous
import jax
import jax.numpy as jnp
from jax.experimental import pallas as pl
from jax.experimental.pallas import tpu as pltpu

_E = 64
_K = 8
_ALPHA = 0.01


def _router_block(x_ref, wt_ref, tw_ref, ti_ref, aux_ref, ssum_ref, cnt_ref):
    i = pl.program_id(0)
    n = pl.num_programs(0)

    x = x_ref[...]
    wt = wt_ref[...]
    logits = jnp.dot(x, wt, preferred_element_type=jnp.float32)

    m = jnp.max(logits, axis=1, keepdims=True)
    ex = jnp.exp(logits - m)
    z = jnp.sum(ex, axis=1, keepdims=True)
    scores = (ex / z).T

    iota_f = jax.lax.broadcasted_iota(jnp.int32, scores.shape, 0).astype(
        jnp.float32)
    sbits = jax.lax.bitcast_convert_type(scores, jnp.int32)
    work = sbits
    mks = []
    idxs = []
    for _ in range(_K):
        mk = jnp.max(work, axis=0, keepdims=True)
        eq = work == mk
        idxs.append(jnp.sum(jnp.where(eq, iota_f, 0.0), axis=0,
                            keepdims=True))
        work = jnp.where(eq, jnp.int32(-(2**31)), work)
        mks.append(mk)

    mkcat = jnp.concatenate(mks, axis=0)
    ti = jnp.concatenate(idxs, axis=0).astype(jnp.int32)
    tw = jax.lax.bitcast_convert_type(mkcat, jnp.float32)
    tw = tw / (jnp.sum(tw, axis=0, keepdims=True) + 1e-9)

    tw_ref[...] = tw
    ti_ref[...] = ti

    hits = (sbits >= mks[-1]).astype(jnp.float32)
    block_ssum = jnp.sum(scores, axis=1, keepdims=True)
    block_cnt = jnp.sum(hits, axis=1, keepdims=True)

    @pl.when(i == 0)
    def _init():
        ssum_ref[...] = block_ssum
        cnt_ref[...] = block_cnt

    @pl.when(i > 0)
    def _acc():
        ssum_ref[...] += block_ssum
        cnt_ref[...] += block_cnt

    @pl.when(i == n - 1)
    def _finish():
        t_total = n * x.shape[0]
        scale = _ALPHA * _E / (float(t_total) * float(t_total) * _K)
        s = jnp.sum(ssum_ref[...] * cnt_ref[...], axis=0, keepdims=True)
        aux_ref[...] = s * scale


def kernel(x, W):
    bsz, seq, d = x.shape
    t = bsz * seq
    xf = x.reshape(t, d)
    wt = W.T

    blk = 512
    grid = (t // blk,)

    tw_kt, ti_kt, aux = pl.pallas_call(
        _router_block,
        grid=grid,
        in_specs=[
            pl.BlockSpec((blk, d), lambda i: (i, 0)),
            pl.BlockSpec((d, _E), lambda i: (0, 0)),
        ],
        out_specs=[
            pl.BlockSpec((_K, blk), lambda i: (0, i)),
            pl.BlockSpec((_K, blk), lambda i: (0, i)),
            pl.BlockSpec((1, 1), lambda i: (0, 0)),
        ],
        out_shape=[
            jax.ShapeDtypeStruct((_K, t), jnp.float32),
            jax.ShapeDtypeStruct((_K, t), jnp.int32),
            jax.ShapeDtypeStruct((1, 1), jnp.float32),
        ],
        scratch_shapes=[
            pltpu.VMEM((_E, 1), jnp.float32),
            pltpu.VMEM((_E, 1), jnp.float32),
        ],
    )(xf, wt)

    return tw_kt.T, ti_kt.T, aux[0, 0]


# speedup vs baseline: 1.9558x; 1.0430x over previous
import jax
import jax.numpy as jnp
from jax.experimental import pallas as pl
from jax.experimental.pallas import tpu as pltpu

_E = 64
_K = 8
_ALPHA = 0.01


def _router_block(x_ref, wt_ref, tw_ref, ti_ref, aux_ref, ssum_ref, cnt_ref):
    i = pl.program_id(0)
    n = pl.num_programs(0)

    x = x_ref[...]
    wt = wt_ref[...]
    logits = jnp.dot(x, wt, preferred_element_type=jnp.float32)

    m = jnp.max(logits, axis=1, keepdims=True)
    ex = jnp.exp(logits - m)
    z = jnp.sum(ex, axis=1, keepdims=True)
    scores = (ex / z).T

    iota_f = jax.lax.broadcasted_iota(jnp.int32, scores.shape, 0).astype(
        jnp.float32)
    sbits = jax.lax.bitcast_convert_type(scores, jnp.int32)
    work = sbits
    mks = []
    idxs = []
    for _ in range(_K):
        mk = jnp.max(work, axis=0, keepdims=True)
        eq = work == mk
        idxs.append(jnp.sum(jnp.where(eq, iota_f, 0.0), axis=0,
                            keepdims=True))
        work = jnp.where(eq, jnp.int32(-(2**31)), work)
        mks.append(mk)

    mkcat = jnp.concatenate(mks, axis=0)
    ti = jnp.concatenate(idxs, axis=0).astype(jnp.int32)
    tw = jax.lax.bitcast_convert_type(mkcat, jnp.float32)
    tw = tw / (jnp.sum(tw, axis=0, keepdims=True) + 1e-9)

    tw_ref[...] = tw
    ti_ref[...] = ti

    hits = (sbits >= mks[-1]).astype(jnp.float32)
    block_ssum = jnp.sum(scores, axis=1, keepdims=True)
    block_cnt = jnp.sum(hits, axis=1, keepdims=True)

    @pl.when(i == 0)
    def _init():
        ssum_ref[...] = block_ssum
        cnt_ref[...] = block_cnt

    @pl.when(i > 0)
    def _acc():
        ssum_ref[...] += block_ssum
        cnt_ref[...] += block_cnt

    @pl.when(i == n - 1)
    def _finish():
        t_total = n * x.shape[0]
        scale = _ALPHA * _E / (float(t_total) * float(t_total) * _K)
        s = jnp.sum(ssum_ref[...] * cnt_ref[...], axis=0, keepdims=True)
        aux_ref[...] = s * scale


def kernel(x, W):
    bsz, seq, d = x.shape
    t = bsz * seq
    xf = x.reshape(t, d)
    wt = W.T

    blk = 1024
    grid = (t // blk,)

    tw_kt, ti_kt, aux = pl.pallas_call(
        _router_block,
        grid=grid,
        in_specs=[
            pl.BlockSpec((blk, d), lambda i: (i, 0)),
            pl.BlockSpec((d, _E), lambda i: (0, 0)),
        ],
        out_specs=[
            pl.BlockSpec((_K, blk), lambda i: (0, i)),
            pl.BlockSpec((_K, blk), lambda i: (0, i)),
            pl.BlockSpec((1, 1), lambda i: (0, 0)),
        ],
        out_shape=[
            jax.ShapeDtypeStruct((_K, t), jnp.float32),
            jax.ShapeDtypeStruct((_K, t), jnp.int32),
            jax.ShapeDtypeStruct((1, 1), jnp.float32),
        ],
        scratch_shapes=[
            pltpu.VMEM((_E, 1), jnp.float32),
            pltpu.VMEM((_E, 1), jnp.float32),
        ],
    )(xf, wt)

    return tw_kt.T, ti_kt.T, aux[0, 0]
